# baseline (speedup 1.0000x reference)
import functools

import jax
import jax.numpy as jnp
from jax import lax
from jax.experimental import pallas as pl
from jax.experimental.pallas import tpu as pltpu
from jax.experimental.pallas import tpu_sc as plsc

N = 10000
D = 128
NE = 320000
B = 64
NC = 2
NS = 16
NW = NC * NS
EPW = NE // NW
C = 120
NCH = 84
NEP = NW * NCH * C
NSINK = 16
NA = N + NSINK
NBUF = 3
NIB = 6
WB = 624
REMZ = NA - NS * WB
REMW = N - NS * WB
RB = 2000
NB = N // RB

@functools.cache
def _sc_agg():
    mesh = plsc.VectorSubcoreMesh(core_axis_name="c", subcore_axis_name="s",
                                  num_cores=NC, num_subcores=NS)
    return pl.kernel(
        _sc_agg_body,
        out_type=jax.ShapeDtypeStruct((NC, N, D), jnp.float32),
        mesh=mesh,
        scratch_types=[
            pltpu.VMEM((NIB, 2, C), jnp.int32),
            pltpu.VMEM((NBUF, C, D), jnp.float32),
            pltpu.VMEM_SHARED((NA, D), jnp.float32),
            [pltpu.SemaphoreType.DMA] * NIB,
            [pltpu.SemaphoreType.DMA] * NBUF,
            pltpu.SemaphoreType.DMA,
        ],
    )


def _sc_agg_body(h_hbm, ei_hbm, out_hbm, idx_v, rows_v, agg_sh, isems, dsems,
                 zsem):
    cid = lax.axis_index("c")
    sid = lax.axis_index("s")
    w = cid * NS + sid

    def _idx_start(s, j):
        pltpu.async_copy(ei_hbm.at[w, j], idx_v.at[s], isems[s])

    def _idx_wait(s, j):
        pltpu.make_async_copy(ei_hbm.at[w, j], idx_v.at[s], isems[s]).wait()

    def _gather(b, s, j):
        pltpu.async_copy(h_hbm.at[idx_v.at[s, 0]], rows_v.at[b], dsems[b])

    def _gather_wait(b, s, j):
        pltpu.make_async_copy(h_hbm.at[idx_v.at[s, 0]], rows_v.at[b],
                              dsems[b]).wait()

    def _scatter(b, s, j):
        pltpu.async_copy(rows_v.at[b], agg_sh.at[idx_v.at[s, 1]], dsems[b],
                         add=True)

    def _scatter_wait(b, s, j):
        pltpu.make_async_copy(rows_v.at[b], agg_sh.at[idx_v.at[s, 1]],
                              dsems[b]).wait()

    for s in range(NIB):
        _idx_start(s, s)
    for j in range(NBUF - 1):
        _idx_wait(j % NIB, j)
        _gather(j % NBUF, j % NIB, j)

    @pl.when(cid == 0)
    def _():
        for t in range(WB // 104):
            pltpu.async_copy(h_hbm.at[pl.ds(sid * WB + t * 104, 104)],
                             agg_sh.at[pl.ds(sid * WB + t * 104, 104)], zsem)

        @pl.when(sid == 0)
        def _():
            pltpu.async_copy(h_hbm.at[pl.ds(NS * WB, REMW)],
                             agg_sh.at[pl.ds(NS * WB, REMW)], zsem)

        for t in range(WB // 104):
            pltpu.make_async_copy(h_hbm.at[pl.ds(sid * WB + t * 104, 104)],
                                  agg_sh.at[pl.ds(sid * WB + t * 104, 104)],
                                  zsem).wait()

        @pl.when(sid == 0)
        def _():
            pltpu.make_async_copy(h_hbm.at[pl.ds(NS * WB, REMW)],
                                  agg_sh.at[pl.ds(NS * WB, REMW)], zsem).wait()

    @pl.when(cid == 1)
    def _():
        zeros16 = jnp.zeros((16,), jnp.float32)

        def _zero_row(r, carry):
            for cc in range(D // 16):
                rows_v[NBUF - 1, r, pl.ds(cc * 16, 16)] = zeros16
            return carry

        lax.fori_loop(0, C, _zero_row, 0)
        zsrc = rows_v.at[NBUF - 1, pl.ds(0, 78)]
        for t in range(WB // 78):
            pltpu.async_copy(zsrc, agg_sh.at[pl.ds(sid * WB + t * 78, 78)],
                             zsem)

        @pl.when(sid == 0)
        def _():
            pltpu.async_copy(rows_v.at[NBUF - 1, pl.ds(0, REMZ)],
                             agg_sh.at[pl.ds(NS * WB, REMZ)], zsem)

        for t in range(WB // 78):
            pltpu.make_async_copy(zsrc,
                                  agg_sh.at[pl.ds(sid * WB + t * 78, 78)],
                                  zsem).wait()

        @pl.when(sid == 0)
        def _():
            pltpu.make_async_copy(rows_v.at[NBUF - 1, pl.ds(0, REMZ)],
                                  agg_sh.at[pl.ds(NS * WB, REMZ)], zsem).wait()

    _idx_wait((NBUF - 1) % NIB, NBUF - 1)
    _gather(NBUF - 1, (NBUF - 1) % NIB, NBUF - 1)
    plsc.subcore_barrier()

    def _ring(g, carry):
        for u in range(NIB):
            jj = g * NIB + u
            b = u % NBUF
            _gather_wait(b, u, jj)
            pltpu.sync_copy(rows_v.at[b], agg_sh.at[idx_v.at[u, 1]], add=True)
            _idx_start(u, jj + NIB)
            _idx_wait((u + NBUF) % NIB, jj + NBUF)
            _gather(b, (u + NBUF) % NIB, jj + NBUF)
        return carry

    lax.fori_loop(0, NCH // NIB - 1, _ring, 0)
    for u in range(NIB):
        jj = NCH - NIB + u
        b = u % NBUF
        _gather_wait(b, u, jj)
        pltpu.sync_copy(rows_v.at[b], agg_sh.at[idx_v.at[u, 1]], add=True)
        if jj + NBUF < NCH:
            _idx_wait((u + NBUF) % NIB, jj + NBUF)
            _gather(b, (u + NBUF) % NIB, jj + NBUF)
    plsc.subcore_barrier()

    pltpu.sync_copy(
        agg_sh.at[pl.ds(sid * WB, WB)],
        out_hbm.at[cid, pl.ds(sid * WB, WB)],
    )

    @pl.when(sid == 0)
    def _():
        pltpu.sync_copy(
            agg_sh.at[pl.ds(NS * WB, REMW)],
            out_hbm.at[cid, pl.ds(NS * WB, REMW)],
        )


def _ln_relu(y, g, gb):
    mu = jnp.mean(y, axis=-1, keepdims=True)
    d = y - mu
    var = jnp.mean(d * d, axis=-1, keepdims=True)
    yn = d * lax.rsqrt(var + 1e-5) * g + gb
    return jnp.maximum(yn, 0.0)


def _mlp_body(aggp, w1, b1, g1, gb1, w2, b2, g2, gb2, out):
    z = aggp[0] + aggp[1]
    y = jnp.dot(z, w1[...], preferred_element_type=jnp.float32) + b1[...]
    y = _ln_relu(y, g1[...], gb1[...])
    y = jnp.dot(y, w2[...], preferred_element_type=jnp.float32) + b2[...]
    out[...] = _ln_relu(y, g2[...], gb2[...])


_row_spec = pl.BlockSpec((RB, D), lambda i: (i, 0))
_mat_spec = pl.BlockSpec((D, D), lambda i: (0, 0))
_vec_spec = pl.BlockSpec((1, D), lambda i: (0, 0))

_mlp_call = pl.pallas_call(
    _mlp_body,
    grid=(NB,),
    in_specs=[
        pl.BlockSpec((NC, RB, D), lambda i: (0, i, 0)),
        _mat_spec, _vec_spec, _vec_spec, _vec_spec,
        _mat_spec, _vec_spec, _vec_spec, _vec_spec,
    ],
    out_specs=_row_spec,
    out_shape=jax.ShapeDtypeStruct((N, D), jnp.float32),
)


def _readout_body(aggp, h0, h1, h2, w1, b1, g1, gb1, w2, b2, g2, gb2,
                  bat, r0, r1, r2, r3, bsum, out, acc):
    i = pl.program_id(0)

    @pl.when(i == 0)
    def _():
        acc[...] = jnp.zeros_like(acc)

    z = aggp[0] + aggp[1]
    y = jnp.dot(z, w1[...], preferred_element_type=jnp.float32) + b1[...]
    y = _ln_relu(y, g1[...], gb1[...])
    y = jnp.dot(y, w2[...], preferred_element_type=jnp.float32) + b2[...]
    h3 = _ln_relu(y, g2[...], gb2[...])

    gid = lax.broadcasted_iota(jnp.int32, (B, RB), 0)
    onehot = (bat[0, :, :] == gid).astype(jnp.float32)
    for k, h in enumerate((h0[...], h1[...], h2[...], h3)):
        acc[k] += jnp.dot(onehot, h, preferred_element_type=jnp.float32)

    @pl.when(i == NB - 1)
    def _():
        score = bsum[...]
        for k, w in enumerate((r0, r1, r2, r3)):
            score = score + jnp.dot(acc[k], w[...],
                                    preferred_element_type=jnp.float32)
        out[...] = score


_readout_call = pl.pallas_call(
    _readout_body,
    grid=(NB,),
    in_specs=[
        pl.BlockSpec((NC, RB, D), lambda i: (0, i, 0)),
        _row_spec, _row_spec, _row_spec,
        _mat_spec, _vec_spec, _vec_spec, _vec_spec,
        _mat_spec, _vec_spec, _vec_spec, _vec_spec,
        pl.BlockSpec((1, 1, RB), lambda i: (i, 0, 0)),
        _mat_spec, _mat_spec, _mat_spec, _mat_spec,
        pl.BlockSpec((B, D), lambda i: (0, 0)),
    ],
    out_specs=pl.BlockSpec((B, D), lambda i: (0, 0)),
    out_shape=jax.ShapeDtypeStruct((B, D), jnp.float32),
    scratch_shapes=[pltpu.VMEM((4, B, D), jnp.float32)],
)


def kernel(x, edge_index, batch, conv_params, readout_params):
    pad = NEP - NE
    pad_src = (jnp.arange(pad, dtype=jnp.int32) * 97) % N
    pad_dst = N + (jnp.arange(pad, dtype=jnp.int32) % NSINK)
    srcp = jnp.concatenate([edge_index[0], pad_src]).reshape(NW, NCH, C)
    dstp = jnp.concatenate([edge_index[1], pad_dst]).reshape(NW, NCH, C)
    ei = jnp.stack([srcp, dstp], axis=2)
    ei = jax.lax.optimization_barrier(ei)
    outputs = [x]
    h = x
    for layers in conv_params[:-1]:
        aggp = _sc_agg()(h, ei)
        (w1, b1, g1, gb1), (w2, b2, g2, gb2) = layers
        h = _mlp_call(aggp,
                      w1, b1.reshape(1, D), g1.reshape(1, D), gb1.reshape(1, D),
                      w2, b2.reshape(1, D), g2.reshape(1, D), gb2.reshape(1, D))
        outputs.append(h)
    aggp = _sc_agg()(h, ei)
    (w1, b1, g1, gb1), (w2, b2, g2, gb2) = conv_params[-1]
    bat3 = batch.reshape(NB, 1, RB)
    ws = [w for (w, _) in readout_params]
    bsum = sum(b for (_, b) in readout_params)
    bsum = jnp.broadcast_to(bsum.reshape(1, D), (B, D))
    return _readout_call(aggp, outputs[0], outputs[1], outputs[2],
                         w1, b1.reshape(1, D), g1.reshape(1, D), gb1.reshape(1, D),
                         w2, b2.reshape(1, D), g2.reshape(1, D), gb2.reshape(1, D),
                         bat3, ws[0], ws[1], ws[2], ws[3], bsum)


# speedup vs baseline: 1.0019x; 1.0019x over previous
import functools

import jax
import jax.numpy as jnp
from jax import lax
from jax.experimental import pallas as pl
from jax.experimental.pallas import tpu as pltpu
from jax.experimental.pallas import tpu_sc as plsc

N = 10000
D = 128
NE = 320000
B = 64
NC = 2
NS = 16
NW = NC * NS
EPW = NE // NW
C = 120
NCH = 84
NEP = NW * NCH * C
NSINK = 16
NA = N + NSINK
NBUF = 3
NIB = 6
WB = 624
REMZ = NA - NS * WB
REMW = N - NS * WB
RB = 2000
NB = N // RB

@functools.cache
def _sc_agg():
    mesh = plsc.VectorSubcoreMesh(core_axis_name="c", subcore_axis_name="s",
                                  num_cores=NC, num_subcores=NS)
    return pl.kernel(
        _sc_agg_body,
        out_type=jax.ShapeDtypeStruct((NC, N, D), jnp.float32),
        mesh=mesh,
        scratch_types=[
            pltpu.VMEM((NIB, 2, C), jnp.int32),
            pltpu.VMEM((NBUF, C, D), jnp.float32),
            pltpu.VMEM_SHARED((NA, D), jnp.float32),
            [pltpu.SemaphoreType.DMA] * NIB,
            [pltpu.SemaphoreType.DMA] * NBUF,
            pltpu.SemaphoreType.DMA,
        ],
    )


def _sc_agg_body(h_hbm, ei_hbm, out_hbm, idx_v, rows_v, agg_sh, isems, dsems,
                 zsem):
    cid = lax.axis_index("c")
    sid = lax.axis_index("s")
    w = cid * NS + sid

    def _idx_start(s, j):
        pltpu.async_copy(ei_hbm.at[w, j], idx_v.at[s], isems[s])

    def _idx_wait(s, j):
        pltpu.make_async_copy(ei_hbm.at[w, j], idx_v.at[s], isems[s]).wait()

    def _gather(b, s, j):
        pltpu.async_copy(h_hbm.at[idx_v.at[s, 0]], rows_v.at[b], dsems[b])

    def _gather_wait(b, s, j):
        pltpu.make_async_copy(h_hbm.at[idx_v.at[s, 0]], rows_v.at[b],
                              dsems[b]).wait()

    def _scatter(b, s, j):
        pltpu.async_copy(rows_v.at[b], agg_sh.at[idx_v.at[s, 1]], dsems[b],
                         add=True)

    def _scatter_wait(b, s, j):
        pltpu.make_async_copy(rows_v.at[b], agg_sh.at[idx_v.at[s, 1]],
                              dsems[b]).wait()

    for s in range(NIB):
        _idx_start(s, s)
    for j in range(NBUF - 1):
        _idx_wait(j % NIB, j)
        _gather(j % NBUF, j % NIB, j)

    @pl.when(cid == 0)
    def _():
        for t in range(WB // 104):
            pltpu.async_copy(h_hbm.at[pl.ds(sid * WB + t * 104, 104)],
                             agg_sh.at[pl.ds(sid * WB + t * 104, 104)], zsem)

        @pl.when(sid == 0)
        def _():
            pltpu.async_copy(h_hbm.at[pl.ds(NS * WB, REMW)],
                             agg_sh.at[pl.ds(NS * WB, REMW)], zsem)

        for t in range(WB // 104):
            pltpu.make_async_copy(h_hbm.at[pl.ds(sid * WB + t * 104, 104)],
                                  agg_sh.at[pl.ds(sid * WB + t * 104, 104)],
                                  zsem).wait()

        @pl.when(sid == 0)
        def _():
            pltpu.make_async_copy(h_hbm.at[pl.ds(NS * WB, REMW)],
                                  agg_sh.at[pl.ds(NS * WB, REMW)], zsem).wait()

    @pl.when(cid == 1)
    def _():
        zeros16 = jnp.zeros((16,), jnp.float32)

        def _zero_row(r, carry):
            for cc in range(D // 16):
                rows_v[NBUF - 1, r, pl.ds(cc * 16, 16)] = zeros16
            return carry

        lax.fori_loop(0, C, _zero_row, 0)
        zsrc = rows_v.at[NBUF - 1, pl.ds(0, 78)]
        for t in range(WB // 78):
            pltpu.async_copy(zsrc, agg_sh.at[pl.ds(sid * WB + t * 78, 78)],
                             zsem)

        @pl.when(sid == 0)
        def _():
            pltpu.async_copy(rows_v.at[NBUF - 1, pl.ds(0, REMZ)],
                             agg_sh.at[pl.ds(NS * WB, REMZ)], zsem)

        for t in range(WB // 78):
            pltpu.make_async_copy(zsrc,
                                  agg_sh.at[pl.ds(sid * WB + t * 78, 78)],
                                  zsem).wait()

        @pl.when(sid == 0)
        def _():
            pltpu.make_async_copy(rows_v.at[NBUF - 1, pl.ds(0, REMZ)],
                                  agg_sh.at[pl.ds(NS * WB, REMZ)], zsem).wait()

    _idx_wait((NBUF - 1) % NIB, NBUF - 1)
    _gather(NBUF - 1, (NBUF - 1) % NIB, NBUF - 1)
    plsc.subcore_barrier()

    def _ring(g, carry):
        for u in range(NIB):
            jj = g * NIB + u
            b = u % NBUF
            _gather_wait(b, u, jj)
            pltpu.sync_copy(rows_v.at[b], agg_sh.at[idx_v.at[u, 1]], add=True)
            _idx_start(u, jj + NIB)
            _idx_wait((u + NBUF) % NIB, jj + NBUF)
            _gather(b, (u + NBUF) % NIB, jj + NBUF)
        return carry

    lax.fori_loop(0, NCH // NIB - 1, _ring, 0)
    for u in range(NIB):
        jj = NCH - NIB + u
        b = u % NBUF
        _gather_wait(b, u, jj)
        pltpu.sync_copy(rows_v.at[b], agg_sh.at[idx_v.at[u, 1]], add=True)
        if jj + NBUF < NCH:
            _idx_wait((u + NBUF) % NIB, jj + NBUF)
            _gather(b, (u + NBUF) % NIB, jj + NBUF)
    plsc.subcore_barrier()

    pltpu.sync_copy(
        agg_sh.at[pl.ds(sid * WB, WB)],
        out_hbm.at[cid, pl.ds(sid * WB, WB)],
    )

    @pl.when(sid == 0)
    def _():
        pltpu.sync_copy(
            agg_sh.at[pl.ds(NS * WB, REMW)],
            out_hbm.at[cid, pl.ds(NS * WB, REMW)],
        )


def _ln_relu(y, g, gb):
    mu = jnp.mean(y, axis=-1, keepdims=True)
    d = y - mu
    var = jnp.mean(d * d, axis=-1, keepdims=True)
    yn = d * lax.rsqrt(var + 1e-5) * g + gb
    return jnp.maximum(yn, 0.0)


def _mlp_body(aggp, w1, b1, g1, gb1, w2, b2, g2, gb2, out):
    z = aggp[0] + aggp[1]
    y = jnp.dot(z, w1[...], preferred_element_type=jnp.float32) + b1[...]
    y = _ln_relu(y, g1[...], gb1[...])
    y = jnp.dot(y, w2[...], preferred_element_type=jnp.float32) + b2[...]
    out[...] = _ln_relu(y, g2[...], gb2[...])


_row_spec = pl.BlockSpec((RB, D), lambda i: (i, 0))
_mat_spec = pl.BlockSpec((D, D), lambda i: (0, 0))
_vec_spec = pl.BlockSpec((1, D), lambda i: (0, 0))

_mlp_call = pl.pallas_call(
    _mlp_body,
    grid=(NB,),
    in_specs=[
        pl.BlockSpec((NC, RB, D), lambda i: (0, i, 0)),
        _mat_spec, _vec_spec, _vec_spec, _vec_spec,
        _mat_spec, _vec_spec, _vec_spec, _vec_spec,
    ],
    out_specs=_row_spec,
    out_shape=jax.ShapeDtypeStruct((N, D), jnp.float32),
)


def _readout_body(aggp, h0, h1, h2, w1, b1, g1, gb1, w2, b2, g2, gb2,
                  bat, r0, r1, r2, r3, bsum, out, acc):
    i = pl.program_id(0)

    @pl.when(i == 0)
    def _():
        acc[...] = jnp.zeros_like(acc)

    z = aggp[0] + aggp[1]
    y = jnp.dot(z, w1[...], preferred_element_type=jnp.float32) + b1[...]
    y = _ln_relu(y, g1[...], gb1[...])
    y = jnp.dot(y, w2[...], preferred_element_type=jnp.float32) + b2[...]
    h3 = _ln_relu(y, g2[...], gb2[...])

    gid = lax.broadcasted_iota(jnp.int32, (B, RB), 0)
    onehot = (bat[0, :, :] == gid).astype(jnp.float32)
    for k, h in enumerate((h0[...], h1[...], h2[...], h3)):
        acc[k] += jnp.dot(onehot, h, preferred_element_type=jnp.float32)

    @pl.when(i == NB - 1)
    def _():
        score = bsum[...]
        for k, w in enumerate((r0, r1, r2, r3)):
            score = score + jnp.dot(acc[k], w[...],
                                    preferred_element_type=jnp.float32)
        out[...] = score


_readout_call = pl.pallas_call(
    _readout_body,
    grid=(NB,),
    in_specs=[
        pl.BlockSpec((NC, RB, D), lambda i: (0, i, 0)),
        _row_spec, _row_spec, _row_spec,
        _mat_spec, _vec_spec, _vec_spec, _vec_spec,
        _mat_spec, _vec_spec, _vec_spec, _vec_spec,
        pl.BlockSpec((1, 1, RB), lambda i: (i, 0, 0)),
        _mat_spec, _mat_spec, _mat_spec, _mat_spec,
        pl.BlockSpec((B, D), lambda i: (0, 0)),
    ],
    out_specs=pl.BlockSpec((B, D), lambda i: (0, 0)),
    out_shape=jax.ShapeDtypeStruct((B, D), jnp.float32),
    scratch_shapes=[pltpu.VMEM((4, B, D), jnp.float32)],
)


def kernel(x, edge_index, batch, conv_params, readout_params):
    pad = NEP - NE
    pad_src = (jnp.arange(pad, dtype=jnp.int32) * 97) % N
    pad_dst = N + (jnp.arange(pad, dtype=jnp.int32) % NSINK)
    srcp = jnp.concatenate([edge_index[0], pad_src]).reshape(NW, NCH, C)
    dstp = jnp.concatenate([edge_index[1], pad_dst]).reshape(NW, NCH, C)
    ei = jnp.stack([srcp, dstp], axis=2)
    outputs = [x]
    h = x
    for layers in conv_params[:-1]:
        aggp = _sc_agg()(h, ei)
        (w1, b1, g1, gb1), (w2, b2, g2, gb2) = layers
        h = _mlp_call(aggp,
                      w1, b1.reshape(1, D), g1.reshape(1, D), gb1.reshape(1, D),
                      w2, b2.reshape(1, D), g2.reshape(1, D), gb2.reshape(1, D))
        outputs.append(h)
    aggp = _sc_agg()(h, ei)
    (w1, b1, g1, gb1), (w2, b2, g2, gb2) = conv_params[-1]
    bat3 = batch.reshape(NB, 1, RB)
    ws = [w for (w, _) in readout_params]
    bsum = sum(b for (_, b) in readout_params)
    bsum = jnp.broadcast_to(bsum.reshape(1, D), (B, D))
    return _readout_call(aggp, outputs[0], outputs[1], outputs[2],
                         w1, b1.reshape(1, D), g1.reshape(1, D), gb1.reshape(1, D),
                         w2, b2.reshape(1, D), g2.reshape(1, D), gb2.reshape(1, D),
                         bat3, ws[0], ws[1], ws[2], ws[3], bsum)


# speedup vs baseline: 1.0240x; 1.0221x over previous
import functools

import jax
import jax.numpy as jnp
from jax import lax
from jax.experimental import pallas as pl
from jax.experimental.pallas import tpu as pltpu
from jax.experimental.pallas import tpu_sc as plsc

N = 10000
D = 128
NE = 320000
B = 64
NC = 2
NS = 16
NW = NC * NS
EPW = NE // NW
C = 120
NCH = 84
NEP = NW * NCH * C
NSINK = 16
NA = N + NSINK
NBUF = 3
NIB = 6
WB = 624
REMZ = NA - NS * WB
REMW = N - NS * WB
RB = 2000
NB = N // RB

@functools.cache
def _sc_agg():
    mesh = plsc.VectorSubcoreMesh(core_axis_name="c", subcore_axis_name="s",
                                  num_cores=NC, num_subcores=NS)
    return pl.kernel(
        _sc_agg_body,
        out_type=jax.ShapeDtypeStruct((NC, N, D), jnp.float32),
        mesh=mesh,
        scratch_types=[
            pltpu.VMEM((NIB, 2, C), jnp.int32),
            pltpu.VMEM((NBUF, C, D), jnp.float32),
            pltpu.VMEM_SHARED((NA, D), jnp.float32),
            [pltpu.SemaphoreType.DMA] * NIB,
            [pltpu.SemaphoreType.DMA] * NBUF,
            pltpu.SemaphoreType.DMA,
        ],
    )


def _sc_agg_body(h_hbm, ei_hbm, out_hbm, idx_v, rows_v, agg_sh, isems, dsems,
                 zsem):
    cid = lax.axis_index("c")
    sid = lax.axis_index("s")
    w = cid * NS + sid

    def _idx_start(s, j):
        pltpu.async_copy(ei_hbm.at[w, j], idx_v.at[s], isems[s])

    def _idx_wait(s, j):
        pltpu.make_async_copy(ei_hbm.at[w, j], idx_v.at[s], isems[s]).wait()

    def _gather(b, s, j):
        pltpu.async_copy(h_hbm.at[idx_v.at[s, 0]], rows_v.at[b], dsems[b])

    def _gather_wait(b, s, j):
        pltpu.make_async_copy(h_hbm.at[idx_v.at[s, 0]], rows_v.at[b],
                              dsems[b]).wait()

    def _scatter(b, s, j):
        pltpu.async_copy(rows_v.at[b], agg_sh.at[idx_v.at[s, 1]], dsems[b],
                         add=True)

    def _scatter_wait(b, s, j):
        pltpu.make_async_copy(rows_v.at[b], agg_sh.at[idx_v.at[s, 1]],
                              dsems[b]).wait()

    for s in range(NIB):
        _idx_start(s, s)
    for j in range(NBUF - 1):
        _idx_wait(j % NIB, j)
        _gather(j % NBUF, j % NIB, j)

    zeros16 = jnp.zeros((16,), jnp.float32)

    def _zero_row(r, carry):
        for cc in range(D // 16):
            rows_v[NBUF - 1, r, pl.ds(cc * 16, 16)] = zeros16
        return carry

    lax.fori_loop(0, C, _zero_row, 0)
    zsrc = rows_v.at[NBUF - 1, pl.ds(0, 78)]
    for t in range(WB // 78):
        pltpu.async_copy(zsrc, agg_sh.at[pl.ds(sid * WB + t * 78, 78)], zsem)

    @pl.when(sid == 0)
    def _():
        pltpu.async_copy(rows_v.at[NBUF - 1, pl.ds(0, REMZ)],
                         agg_sh.at[pl.ds(NS * WB, REMZ)], zsem)

    for t in range(WB // 78):
        pltpu.make_async_copy(zsrc, agg_sh.at[pl.ds(sid * WB + t * 78, 78)],
                              zsem).wait()

    @pl.when(sid == 0)
    def _():
        pltpu.make_async_copy(rows_v.at[NBUF - 1, pl.ds(0, REMZ)],
                              agg_sh.at[pl.ds(NS * WB, REMZ)], zsem).wait()

    _idx_wait((NBUF - 1) % NIB, NBUF - 1)
    _gather(NBUF - 1, (NBUF - 1) % NIB, NBUF - 1)
    plsc.subcore_barrier()

    def _ring(g, carry):
        for u in range(NIB):
            jj = g * NIB + u
            b = u % NBUF
            _gather_wait(b, u, jj)
            pltpu.sync_copy(rows_v.at[b], agg_sh.at[idx_v.at[u, 1]], add=True)
            _idx_start(u, jj + NIB)
            _idx_wait((u + NBUF) % NIB, jj + NBUF)
            _gather(b, (u + NBUF) % NIB, jj + NBUF)
        return carry

    lax.fori_loop(0, NCH // NIB - 1, _ring, 0)
    for u in range(NIB):
        jj = NCH - NIB + u
        b = u % NBUF
        _gather_wait(b, u, jj)
        pltpu.sync_copy(rows_v.at[b], agg_sh.at[idx_v.at[u, 1]], add=True)
        if jj + NBUF < NCH:
            _idx_wait((u + NBUF) % NIB, jj + NBUF)
            _gather(b, (u + NBUF) % NIB, jj + NBUF)
    plsc.subcore_barrier()

    pltpu.sync_copy(
        agg_sh.at[pl.ds(sid * WB, WB)],
        out_hbm.at[cid, pl.ds(sid * WB, WB)],
    )

    @pl.when(sid == 0)
    def _():
        pltpu.sync_copy(
            agg_sh.at[pl.ds(NS * WB, REMW)],
            out_hbm.at[cid, pl.ds(NS * WB, REMW)],
        )


def _ln_relu(y, g, gb):
    mu = jnp.mean(y, axis=-1, keepdims=True)
    d = y - mu
    var = jnp.mean(d * d, axis=-1, keepdims=True)
    yn = d * lax.rsqrt(var + 1e-5) * g + gb
    return jnp.maximum(yn, 0.0)


def _mlp_body(aggp, h, w1, b1, g1, gb1, w2, b2, g2, gb2, out):
    z = aggp[0] + aggp[1] + h[...]
    y = jnp.dot(z, w1[...], preferred_element_type=jnp.float32) + b1[...]
    y = _ln_relu(y, g1[...], gb1[...])
    y = jnp.dot(y, w2[...], preferred_element_type=jnp.float32) + b2[...]
    out[...] = _ln_relu(y, g2[...], gb2[...])


_row_spec = pl.BlockSpec((RB, D), lambda i: (i, 0))
_mat_spec = pl.BlockSpec((D, D), lambda i: (0, 0))
_vec_spec = pl.BlockSpec((1, D), lambda i: (0, 0))

_mlp_call = pl.pallas_call(
    _mlp_body,
    grid=(NB,),
    in_specs=[
        pl.BlockSpec((NC, RB, D), lambda i: (0, i, 0)),
        _row_spec,
        _mat_spec, _vec_spec, _vec_spec, _vec_spec,
        _mat_spec, _vec_spec, _vec_spec, _vec_spec,
    ],
    out_specs=_row_spec,
    out_shape=jax.ShapeDtypeStruct((N, D), jnp.float32),
)


def _readout_body(aggp, h0, h1, h2, w1, b1, g1, gb1, w2, b2, g2, gb2,
                  bat, r0, r1, r2, r3, bsum, out, acc):
    i = pl.program_id(0)

    @pl.when(i == 0)
    def _():
        acc[...] = jnp.zeros_like(acc)

    z = aggp[0] + aggp[1] + h2[...]
    y = jnp.dot(z, w1[...], preferred_element_type=jnp.float32) + b1[...]
    y = _ln_relu(y, g1[...], gb1[...])
    y = jnp.dot(y, w2[...], preferred_element_type=jnp.float32) + b2[...]
    h3 = _ln_relu(y, g2[...], gb2[...])

    gid = lax.broadcasted_iota(jnp.int32, (B, RB), 0)
    onehot = (bat[0, :, :] == gid).astype(jnp.float32)
    for k, h in enumerate((h0[...], h1[...], h2[...], h3)):
        acc[k] += jnp.dot(onehot, h, preferred_element_type=jnp.float32)

    @pl.when(i == NB - 1)
    def _():
        score = bsum[...]
        for k, w in enumerate((r0, r1, r2, r3)):
            score = score + jnp.dot(acc[k], w[...],
                                    preferred_element_type=jnp.float32)
        out[...] = score


_readout_call = pl.pallas_call(
    _readout_body,
    grid=(NB,),
    in_specs=[
        pl.BlockSpec((NC, RB, D), lambda i: (0, i, 0)),
        _row_spec, _row_spec, _row_spec,
        _mat_spec, _vec_spec, _vec_spec, _vec_spec,
        _mat_spec, _vec_spec, _vec_spec, _vec_spec,
        pl.BlockSpec((1, 1, RB), lambda i: (i, 0, 0)),
        _mat_spec, _mat_spec, _mat_spec, _mat_spec,
        pl.BlockSpec((B, D), lambda i: (0, 0)),
    ],
    out_specs=pl.BlockSpec((B, D), lambda i: (0, 0)),
    out_shape=jax.ShapeDtypeStruct((B, D), jnp.float32),
    scratch_shapes=[pltpu.VMEM((4, B, D), jnp.float32)],
)


def kernel(x, edge_index, batch, conv_params, readout_params):
    pad = NEP - NE
    pad_src = (jnp.arange(pad, dtype=jnp.int32) * 97) % N
    pad_dst = N + (jnp.arange(pad, dtype=jnp.int32) % NSINK)
    srcp = jnp.concatenate([edge_index[0], pad_src]).reshape(NW, NCH, C)
    dstp = jnp.concatenate([edge_index[1], pad_dst]).reshape(NW, NCH, C)
    ei = jnp.stack([srcp, dstp], axis=2)
    outputs = [x]
    h = x
    for layers in conv_params[:-1]:
        aggp = _sc_agg()(h, ei)
        (w1, b1, g1, gb1), (w2, b2, g2, gb2) = layers
        h = _mlp_call(aggp, h,
                      w1, b1.reshape(1, D), g1.reshape(1, D), gb1.reshape(1, D),
                      w2, b2.reshape(1, D), g2.reshape(1, D), gb2.reshape(1, D))
        outputs.append(h)
    aggp = _sc_agg()(h, ei)
    (w1, b1, g1, gb1), (w2, b2, g2, gb2) = conv_params[-1]
    bat3 = batch.reshape(NB, 1, RB)
    ws = [w for (w, _) in readout_params]
    bsum = sum(b for (_, b) in readout_params)
    bsum = jnp.broadcast_to(bsum.reshape(1, D), (B, D))
    return _readout_call(aggp, outputs[0], outputs[1], outputs[2],
                         w1, b1.reshape(1, D), g1.reshape(1, D), gb1.reshape(1, D),
                         w2, b2.reshape(1, D), g2.reshape(1, D), gb2.reshape(1, D),
                         bat3, ws[0], ws[1], ws[2], ws[3], bsum)
